# baseline (speedup 1.0000x reference)
import jax
import jax.numpy as jnp
from jax import lax
from jax.experimental import pallas as pl
from jax.experimental.pallas import tpu as pltpu
from jax.experimental.pallas import tpu_sc as plsc

VOCAB = 1000000
EMBED_DIM = 64
BATCH = 16384
HIST = 50

NC, NS, L = 2, 16, 16
NW = NC * NS
B_TOTAL = BATCH * HIST
PER_W = B_TOTAL // NW

BBLK = 128
CB_PER_W = BATCH // BBLK // NW
HP = HIST // 2
UNITS = CB_PER_W * HP


NBLK = VOCAB // 128
BLK_PER_W = NBLK // NW


def _pack_body(tabT_h, out_h, a0, a1, b0, b1, semr, semw):
    wid = lax.axis_index("s") * NC + lax.axis_index("c")
    lane = lax.iota(jnp.int32, L)
    vprow = [g * 8 + (lane >> 1) for g in range(128 // L)]
    t64 = (lane & 1) * EMBED_DIM
    c0 = wid * BLK_PER_W

    def fire_read(t, av):
        pltpu.async_copy(tabT_h.at[:, pl.ds((c0 + t) * 128, 128)], av, semr)

    def drain_read(av):
        pltpu.make_async_copy(tabT_h.at[:, pl.ds(0, 128)], av, semr).wait()

    def transpose(av, bv):
        @pl.loop(0, EMBED_DIM // 4)
        def _(fg):
            for df in range(4):
                f = fg * 4 + df
                colv = t64 + f
                for g in range(128 // L):
                    vals = av[f, pl.ds(g * L, L)]
                    plsc.store_scatter(bv, [vprow[g], colv], vals)

    def fire_write(t, bv):
        pltpu.async_copy(bv.at[:, pl.ds(0, 128)],
                         out_h.at[pl.ds((c0 + t) * 64, 64)], semw)

    def drain_write(bv):
        pltpu.make_async_copy(bv.at[:, pl.ds(0, 128)],
                              out_h.at[pl.ds(0, 64)], semw).wait()

    fire_read(0, a0)

    @pl.loop(0, BLK_PER_W // 2)
    def _(i):
        t = 2 * i
        fire_read(t + 1, a1)
        drain_read(a0)

        @pl.when(i > 0)
        def _():
            drain_write(b0)

        transpose(a0, b0)
        fire_write(t, b0)

        @pl.when(t + 2 < BLK_PER_W)
        def _():
            fire_read(t + 2, a0)

        drain_read(a1)

        @pl.when(i > 0)
        def _():
            drain_write(b1)

        transpose(a1, b1)
        fire_write(t + 1, b1)

    drain_write(b0)
    drain_write(b1)

    @pl.when(wid < 4)
    def _():
        c = NW * BLK_PER_W + wid
        pltpu.sync_copy(tabT_h.at[:, pl.ds(c * 128, 128)], a0)
        transpose(a0, b0)
        pltpu.sync_copy(b0.at[:, pl.ds(0, 128)],
                        out_h.at[pl.ds(c * 64, 64)])


_pack = pl.kernel(
    _pack_body,
    out_type=jax.ShapeDtypeStruct((VOCAB // 2, 2 * EMBED_DIM), jnp.float32),
    mesh=plsc.VectorSubcoreMesh(core_axis_name="c", subcore_axis_name="s"),
    scratch_types=[
        pltpu.VMEM((EMBED_DIM, 128), jnp.float32),
        pltpu.VMEM((EMBED_DIM, 128), jnp.float32),
        pltpu.VMEM((EMBED_DIM, 130), jnp.float32),
        pltpu.VMEM((EMBED_DIM, 130), jnp.float32),
        pltpu.SemaphoreType.DMA,
        pltpu.SemaphoreType.DMA,
    ],
    compiler_params=pltpu.CompilerParams(use_tc_tiling_on_sc=True,
                                         needs_layout_passes=False),
)


def _body(table_h, idx_h, out_h, idx_v, gl0, gl1, g0, g1, tv, sem0, sem1):
    wid = lax.axis_index("s") * NC + lax.axis_index("c")
    pltpu.sync_copy(idx_h.at[pl.ds(wid * PER_W, PER_W)], idx_v)

    lane = lax.iota(jnp.int32, L)
    lane50 = lane * HIST

    def fire(u, gl, gb, sem):
        cbi = u // HP
        hp = u - cbi * HP
        qbase = cbi * (BBLK * HIST) + 2 * hp
        for m in range(BBLK // L):
            qv = qbase + m * (L * HIST) + lane50
            gl[pl.ds(m * L, L)] = plsc.load_gather(idx_v, [qv])
            gl[pl.ds(BBLK + m * L, L)] = plsc.load_gather(idx_v, [qv + 1])
        pltpu.async_copy(table_h.at[gl.at[pl.ds(0, BBLK)]],
                         gb.at[pl.ds(0, BBLK)], sem)
        pltpu.async_copy(table_h.at[gl.at[pl.ds(BBLK, BBLK)]],
                         gb.at[pl.ds(BBLK, BBLK)], sem)

    def drain(gb, sem):
        pltpu.make_async_copy(table_h.at[pl.ds(0, 2 * BBLK)], gb, sem).wait()

    kvec = lane & 7
    rvec = [q * 2 + (lane >> 3) for q in range(EMBED_DIM // L)]
    zero = lane * 0

    def transpose_write(u, gb):
        cbi = u // HP
        hp = u - cbi * HP

        for hh in range(2):
            hv = zero + hh

            @pl.loop(0, BBLK // 4)
            def _(jg):
                for dj in range(4):
                    j = jg * 4 + dj
                    jv = zero + j
                    for q in range(EMBED_DIM // L):
                        vals = gb[hh * BBLK + j, pl.ds(q * L, L)]
                        plsc.store_scatter(tv, [hv, rvec[q], kvec, jv], vals)

        cb = wid * CB_PER_W + cbi
        pltpu.sync_copy(tv.at[0, :, :, pl.ds(0, BBLK)],
                        out_h.at[2 * hp, :, cb])
        pltpu.sync_copy(tv.at[1, :, :, pl.ds(0, BBLK)],
                        out_h.at[2 * hp + 1, :, cb])

    fire(0, gl0, g0, sem0)

    @pl.loop(0, UNITS // 2)
    def _(i):
        u = 2 * i
        fire(u + 1, gl1, g1, sem1)
        drain(g0, sem0)
        transpose_write(u, g0)

        @pl.when(u + 2 < UNITS)
        def _():
            fire(u + 2, gl0, g0, sem0)

        drain(g1, sem1)
        transpose_write(u + 1, g1)


_gather = pl.kernel(
    _body,
    out_type=jax.ShapeDtypeStruct((HIST, 8, BATCH // BBLK, 8, BBLK),
                                  jnp.float32),
    mesh=plsc.VectorSubcoreMesh(core_axis_name="c", subcore_axis_name="s"),
    scratch_types=[
        pltpu.VMEM((PER_W,), jnp.int32),
        pltpu.VMEM((2 * BBLK,), jnp.int32),
        pltpu.VMEM((2 * BBLK,), jnp.int32),
        pltpu.VMEM((2 * BBLK, EMBED_DIM), jnp.float32),
        pltpu.VMEM((2 * BBLK, EMBED_DIM), jnp.float32),
        pltpu.VMEM((2, 8, 8, BBLK + 1), jnp.float32),
        pltpu.SemaphoreType.DMA,
        pltpu.SemaphoreType.DMA,
    ],
    compiler_params=pltpu.CompilerParams(use_tc_tiling_on_sc=False, needs_layout_passes=False),
)


def kernel(input_ids, table):
    packed = _pack(table.T)
    tail = table[VOCAB - 64:].reshape(32, 2 * EMBED_DIM)
    packed = lax.dynamic_update_slice(packed, tail, (VOCAB // 2 - 32, 0))
    tab_lin = packed.reshape(VOCAB, EMBED_DIM)
    idx1 = input_ids.reshape(-1)
    out5 = _gather(tab_lin, idx1)
    return out5.transpose(2, 4, 0, 1, 3).reshape(BATCH, HIST, EMBED_DIM)


# speedup vs baseline: 1.5887x; 1.5887x over previous
import jax
import jax.numpy as jnp
from jax import lax
from jax.experimental import pallas as pl
from jax.experimental.pallas import tpu as pltpu
from jax.experimental.pallas import tpu_sc as plsc

VOCAB = 1000000
EMBED_DIM = 64
BATCH = 16384
HIST = 50

NC, NS, L = 2, 16, 16
NW = NC * NS
B_TOTAL = BATCH * HIST
PER_W = B_TOTAL // NW

BBLK = 128
CB_PER_W = BATCH // BBLK // NW
HP = HIST // 2
UNITS = CB_PER_W * HP


def _body(table_h, idx_h, out_h, idx_v, gl0, gl1, g0, g1, tv, sem0, sem1):
    wid = lax.axis_index("s") * NC + lax.axis_index("c")
    pltpu.sync_copy(idx_h.at[pl.ds(wid * PER_W, PER_W)], idx_v)

    lane = lax.iota(jnp.int32, L)
    lane50 = lane * HIST

    def fire(u, gl, gb, sem):
        cbi = u // HP
        hp = u - cbi * HP
        qbase = cbi * (BBLK * HIST) + 2 * hp
        for m in range(BBLK // L):
            qv = qbase + m * (L * HIST) + lane50
            gl[pl.ds(m * L, L)] = plsc.load_gather(idx_v, [qv])
            gl[pl.ds(BBLK + m * L, L)] = plsc.load_gather(idx_v, [qv + 1])
        pltpu.async_copy(table_h.at[gl.at[pl.ds(0, BBLK)]],
                         gb.at[pl.ds(0, BBLK)], sem)
        pltpu.async_copy(table_h.at[gl.at[pl.ds(BBLK, BBLK)]],
                         gb.at[pl.ds(BBLK, BBLK)], sem)

    def drain(gb, sem):
        pltpu.make_async_copy(table_h.at[pl.ds(0, 2 * BBLK)], gb, sem).wait()

    kvec = lane & 7
    rvec = [q * 2 + (lane >> 3) for q in range(EMBED_DIM // L)]
    zero = lane * 0

    def transpose_write(u, gb):
        cbi = u // HP
        hp = u - cbi * HP

        for hh in range(2):
            hv = zero + hh

            @pl.loop(0, BBLK // 4)
            def _(jg):
                for dj in range(4):
                    j = jg * 4 + dj
                    jv = zero + j
                    for q in range(EMBED_DIM // L):
                        vals = gb[hh * BBLK + j, pl.ds(q * L, L)]
                        plsc.store_scatter(tv, [hv, rvec[q], kvec, jv], vals)

        cb = wid * CB_PER_W + cbi
        pltpu.sync_copy(tv.at[0, :, :, pl.ds(0, BBLK)],
                        out_h.at[2 * hp, :, cb])
        pltpu.sync_copy(tv.at[1, :, :, pl.ds(0, BBLK)],
                        out_h.at[2 * hp + 1, :, cb])

    fire(0, gl0, g0, sem0)

    @pl.loop(0, UNITS // 2)
    def _(i):
        u = 2 * i
        fire(u + 1, gl1, g1, sem1)
        drain(g0, sem0)
        transpose_write(u, g0)

        @pl.when(u + 2 < UNITS)
        def _():
            fire(u + 2, gl0, g0, sem0)

        drain(g1, sem1)
        transpose_write(u + 1, g1)


_gather = pl.kernel(
    _body,
    out_type=jax.ShapeDtypeStruct((HIST, 8, BATCH // BBLK, 8, BBLK),
                                  jnp.float32),
    mesh=plsc.VectorSubcoreMesh(core_axis_name="c", subcore_axis_name="s"),
    scratch_types=[
        pltpu.VMEM((PER_W,), jnp.int32),
        pltpu.VMEM((2 * BBLK,), jnp.int32),
        pltpu.VMEM((2 * BBLK,), jnp.int32),
        pltpu.VMEM((2 * BBLK, EMBED_DIM), jnp.float32),
        pltpu.VMEM((2 * BBLK, EMBED_DIM), jnp.float32),
        pltpu.VMEM((2, 8, 8, BBLK + 1), jnp.float32),
        pltpu.SemaphoreType.DMA,
        pltpu.SemaphoreType.DMA,
    ],
    compiler_params=pltpu.CompilerParams(use_tc_tiling_on_sc=False, needs_layout_passes=False),
)


def kernel(input_ids, table):
    idx1 = input_ids.reshape(-1)
    out5 = _gather(table, idx1)
    return out5.transpose(2, 4, 0, 1, 3).reshape(BATCH, HIST, EMBED_DIM)
